# Initial kernel scaffold; baseline (speedup 1.0000x reference)
import jax
import jax.numpy as jnp
from jax.experimental import pallas as pl


def kernel(x, edge_index, W0, b0, W1, b1):
    raise NotImplementedError("write your pallas kernel here")


# speedup vs baseline: 2.8492x; 2.8492x over previous
import jax
import jax.numpy as jnp
from jax.experimental import pallas as pl


def _mm_kernel(x_ref, w_ref, o_ref):
    o_ref[...] = jnp.dot(x_ref[...], w_ref[...], preferred_element_type=jnp.float32)


def _mm(x, w):
    M, K = x.shape
    _, N = w.shape
    BM = 1000
    return pl.pallas_call(
        _mm_kernel,
        out_shape=jax.ShapeDtypeStruct((M, N), jnp.float32),
        grid=(M // BM,),
        in_specs=[
            pl.BlockSpec((BM, K), lambda i: (i, 0)),
            pl.BlockSpec((K, N), lambda i: (0, 0)),
        ],
        out_specs=pl.BlockSpec((BM, N), lambda i: (i, 0)),
    )(x, w)


def kernel(x, edge_index, W0, b0, W1, b1):
    n = x.shape[0]
    row = edge_index[0]
    col = edge_index[1]
    deg = jnp.ones((n,), dtype=x.dtype).at[col].add(1.0)
    dis = jax.lax.rsqrt(deg)

    def conv(h, W, b):
        xw = _mm(h, W)
        y = xw * dis[:, None]
        acc = jnp.zeros((n, W.shape[1]), dtype=x.dtype).at[col].add(y[row])
        return dis[:, None] * (acc + y) + b

    h = jax.nn.relu(conv(x, W0, b0))
    return conv(h, W1, b1)


# speedup vs baseline: 29.1194x; 10.2200x over previous
import functools

import jax
import jax.numpy as jnp
from jax import lax
from jax.experimental import pallas as pl
from jax.experimental.pallas import tpu as pltpu
from jax.experimental.pallas import tpu_sc as plsc

_INFO = plsc.get_sparse_core_info()
_NC = _INFO.num_cores
_NS = _INFO.num_subcores
_NL = _INFO.num_lanes
_NW = _NC * _NS

_C = 80


def _deg_pass(col3d, npad):
    _, nch, c_w = col3d.shape
    rpt = npad // _NS
    mesh = plsc.VectorSubcoreMesh(core_axis_name="c", subcore_axis_name="s")

    @functools.partial(
        pl.kernel,
        out_type=(jax.ShapeDtypeStruct((npad,), jnp.float32),
                  jax.ShapeDtypeStruct((npad,), jnp.float32)),
        mesh=mesh,
        scratch_types=[
            pltpu.VMEM_SHARED((npad,), jnp.float32),
            pltpu.VMEM((nch, c_w), jnp.int32),
            pltpu.VMEM((c_w,), jnp.float32),
            pltpu.VMEM((rpt,), jnp.float32),
            pltpu.SemaphoreType.DMA,
        ],
    )
    def k(col_hbm, d0_hbm, d1_hbm, deg_sh, cidx, ones_v, zbuf, sem):
        c = lax.axis_index("c")
        s = lax.axis_index("s")
        w = c * _NS + s
        rbase = s * rpt

        def zrow(i, _):
            zbuf[pl.ds(i * _NL, _NL)] = jnp.zeros((_NL,), jnp.float32)
            return 0
        lax.fori_loop(0, rpt // _NL, zrow, 0)
        for t in range(c_w // _NL):
            ones_v[pl.ds(t * _NL, _NL)] = jnp.ones((_NL,), jnp.float32)
        pltpu.sync_copy(zbuf, deg_sh.at[pl.ds(rbase, rpt)])
        pltpu.sync_copy(col_hbm.at[w], cidx)
        plsc.subcore_barrier()

        def fire(k_, _):
            pltpu.async_copy(ones_v, deg_sh.at[cidx.at[k_]], sem, add=True)
            return 0
        lax.fori_loop(0, nch, fire, 0)

        def drain(k_, _):
            pltpu.make_async_copy(ones_v, deg_sh.at[cidx.at[0]], sem).wait()
            return 0
        lax.fori_loop(0, nch, drain, 0)
        plsc.subcore_barrier()

        @pl.when(c == 0)
        def _():
            pltpu.sync_copy(deg_sh.at[pl.ds(rbase, rpt)],
                            d0_hbm.at[pl.ds(rbase, rpt)])

        @pl.when(c != 0)
        def _():
            pltpu.sync_copy(deg_sh.at[pl.ds(rbase, rpt)],
                            d1_hbm.at[pl.ds(rbase, rpt)])

    return k(col3d)


def _edge_pass(y, row4d, col4d):
    npad, d = y.shape
    _, nph, cpp, c_w = row4d.shape
    assert cpp % 2 == 1
    rpt = npad // _NS
    mesh = plsc.VectorSubcoreMesh(core_axis_name="c", subcore_axis_name="s")

    @functools.partial(
        pl.kernel,
        out_type=(jax.ShapeDtypeStruct((npad, d), jnp.float32),
                  jax.ShapeDtypeStruct((npad, d), jnp.float32)),
        mesh=mesh,
        scratch_types=[
            pltpu.VMEM_SHARED((npad, d), jnp.float32),
            pltpu.VMEM((cpp, c_w), jnp.int32),
            pltpu.VMEM((cpp, c_w), jnp.int32),
            pltpu.VMEM((c_w, d), jnp.float32),
            pltpu.VMEM((c_w, d), jnp.float32),
            pltpu.SemaphoreType.DMA,
            pltpu.SemaphoreType.DMA,
        ],
    )
    def k(y_hbm, row_hbm, col_hbm, p0_hbm, p1_hbm,
          acc_sh, ridx, cidx, m0, m1, s0, s1):
        c = lax.axis_index("c")
        s = lax.axis_index("s")
        w = c * _NS + s
        rbase = s * rpt

        @pl.when(c == 0)
        def _():
            pltpu.sync_copy(y_hbm.at[pl.ds(rbase, rpt)],
                            acc_sh.at[pl.ds(rbase, rpt)])

        @pl.when(c != 0)
        def _():
            def zrow(i, _):
                for jj in range(d // _NL):
                    m0[i, pl.ds(jj * _NL, _NL)] = jnp.zeros((_NL,), jnp.float32)
                return 0
            lax.fori_loop(0, c_w, zrow, 0)
            for t in range(rpt // c_w):
                pltpu.sync_copy(m0, acc_sh.at[pl.ds(rbase + t * c_w, c_w)])

        plsc.subcore_barrier()

        def start(k_, mb, sem):
            pltpu.async_copy(y_hbm.at[ridx.at[k_]], mb, sem)

        def wait(mb, sem):
            pltpu.make_async_copy(y_hbm.at[ridx.at[0]], mb, sem).wait()

        def scat(k_, mb):
            pltpu.sync_copy(mb, acc_sh.at[cidx.at[k_]], add=True)

        def phase(p, _):
            pltpu.sync_copy(row_hbm.at[w, p], ridx)
            pltpu.sync_copy(col_hbm.at[w, p], cidx)
            start(0, m0, s0)

            def pair(j, _):
                ka = 2 * j + 1
                start(ka, m1, s1)
                wait(m0, s0)
                scat(ka - 1, m0)
                start(ka + 1, m0, s0)
                wait(m1, s1)
                scat(ka, m1)
                return 0

            lax.fori_loop(0, (cpp - 1) // 2, pair, 0)
            wait(m0, s0)
            scat(cpp - 1, m0)
            return 0

        lax.fori_loop(0, nph, phase, 0)
        plsc.subcore_barrier()

        @pl.when(c == 0)
        def _():
            pltpu.sync_copy(acc_sh.at[pl.ds(rbase, rpt)],
                            p0_hbm.at[pl.ds(rbase, rpt)])

        @pl.when(c != 0)
        def _():
            pltpu.sync_copy(acc_sh.at[pl.ds(rbase, rpt)],
                            p1_hbm.at[pl.ds(rbase, rpt)])

    return k(y, row4d, col4d)


_BM = 1280


def _t1_body(x_ref, w_ref, d0_ref, d1_ref, o_ref):
    dis = lax.rsqrt(d0_ref[...] + d1_ref[...] + 1.0)
    o_ref[...] = dis * jnp.dot(x_ref[...], w_ref[...],
                               preferred_element_type=jnp.float32)


def _t1(x, w, d0, d1):
    npad, d_in = x.shape
    d_out = w.shape[1]
    return pl.pallas_call(
        _t1_body,
        out_shape=jax.ShapeDtypeStruct((npad, d_out), jnp.float32),
        grid=(npad // _BM,),
        in_specs=[
            pl.BlockSpec((_BM, d_in), lambda i: (i, 0)),
            pl.BlockSpec((d_in, d_out), lambda i: (0, 0)),
            pl.BlockSpec((_BM, 1), lambda i: (i, 0)),
            pl.BlockSpec((_BM, 1), lambda i: (i, 0)),
        ],
        out_specs=pl.BlockSpec((_BM, d_out), lambda i: (i, 0)),
    )(x, w, d0, d1)


def _t2_body(p0_ref, p1_ref, d0_ref, d1_ref, b_ref, w_ref, o_ref):
    dis = lax.rsqrt(d0_ref[...] + d1_ref[...] + 1.0)
    h = jnp.maximum(dis * (p0_ref[...] + p1_ref[...]) + b_ref[...], 0.0)
    o_ref[...] = dis * jnp.dot(h, w_ref[...],
                               preferred_element_type=jnp.float32)


def _t2(p0, p1, d0, d1, b, w):
    npad, d = p0.shape
    d_out = w.shape[1]
    return pl.pallas_call(
        _t2_body,
        out_shape=jax.ShapeDtypeStruct((npad, d_out), jnp.float32),
        grid=(npad // _BM,),
        in_specs=[
            pl.BlockSpec((_BM, d), lambda i: (i, 0)),
            pl.BlockSpec((_BM, d), lambda i: (i, 0)),
            pl.BlockSpec((_BM, 1), lambda i: (i, 0)),
            pl.BlockSpec((_BM, 1), lambda i: (i, 0)),
            pl.BlockSpec((1, d), lambda i: (0, 0)),
            pl.BlockSpec((d, d_out), lambda i: (0, 0)),
        ],
        out_specs=pl.BlockSpec((_BM, d_out), lambda i: (i, 0)),
    )(p0, p1, d0, d1, b, w)


def _t3_body(q0_ref, q1_ref, d0_ref, d1_ref, b_ref, o_ref):
    dis = lax.rsqrt(d0_ref[...] + d1_ref[...] + 1.0)
    o_ref[...] = dis * (q0_ref[...] + q1_ref[...]) + b_ref[...]


def _t3(q0, q1, d0, d1, b):
    npad, d = q0.shape
    return pl.pallas_call(
        _t3_body,
        out_shape=jax.ShapeDtypeStruct((npad, d), jnp.float32),
        grid=(npad // _BM,),
        in_specs=[
            pl.BlockSpec((_BM, d), lambda i: (i, 0)),
            pl.BlockSpec((_BM, d), lambda i: (i, 0)),
            pl.BlockSpec((_BM, 1), lambda i: (i, 0)),
            pl.BlockSpec((_BM, 1), lambda i: (i, 0)),
            pl.BlockSpec((1, d), lambda i: (0, 0)),
        ],
        out_specs=pl.BlockSpec((_BM, d), lambda i: (i, 0)),
    )(q0, q1, d0, d1, b)


def kernel(x, edge_index, W0, b0, W1, b1):
    n, _ = x.shape
    e = edge_index.shape[1]
    assert e % (_NW * _C) == 0, (e, _NW, _C)
    grp = _NS * _C
    npad = ((n + grp - 1) // grp) * grp
    assert npad % _BM == 0

    xpad = jnp.pad(x, ((0, npad - n), (0, 0)))
    nch = e // (_NW * _C)
    nph = 5
    assert nch % nph == 0 and (nch // nph) % 2 == 1
    row4d = edge_index[0].reshape(_NW, nph, nch // nph, _C)
    col4d = edge_index[1].reshape(_NW, nph, nch // nph, _C)

    d0, d1 = _deg_pass(col4d.reshape(_NW, nch, _C), npad)
    d0, d1 = d0[:, None], d1[:, None]

    y0 = _t1(xpad, W0, d0, d1)
    p0, p1 = _edge_pass(y0, row4d, col4d)
    y1 = _t2(p0, p1, d0, d1, b0[None, :], W1)
    q0, q1 = _edge_pass(y1, row4d, col4d)
    out = _t3(q0, q1, d0, d1, b1[None, :])
    return out[:n]


# speedup vs baseline: 29.8229x; 1.0242x over previous
import functools

import jax
import jax.numpy as jnp
from jax import lax
from jax.experimental import pallas as pl
from jax.experimental.pallas import tpu as pltpu
from jax.experimental.pallas import tpu_sc as plsc

_INFO = plsc.get_sparse_core_info()
_NC = _INFO.num_cores
_NS = _INFO.num_subcores
_NL = _INFO.num_lanes
_NW = _NC * _NS

_C = 80
_NB = 4


def _deg_pass(col3d, npad):
    _, nch, c_w = col3d.shape
    rpt = npad // _NS
    mesh = plsc.VectorSubcoreMesh(core_axis_name="c", subcore_axis_name="s")

    @functools.partial(
        pl.kernel,
        out_type=(jax.ShapeDtypeStruct((npad,), jnp.float32),
                  jax.ShapeDtypeStruct((npad,), jnp.float32)),
        mesh=mesh,
        scratch_types=[
            pltpu.VMEM_SHARED((npad,), jnp.float32),
            pltpu.VMEM((nch, c_w), jnp.int32),
            pltpu.VMEM((c_w,), jnp.float32),
            pltpu.VMEM((((rpt + _NL - 1) // _NL) * _NL,), jnp.float32),
            pltpu.SemaphoreType.DMA,
        ],
    )
    def k(col_hbm, d0_hbm, d1_hbm, deg_sh, cidx, ones_v, zbuf, sem):
        c = lax.axis_index("c")
        s = lax.axis_index("s")
        w = c * _NS + s
        rbase = s * rpt

        def zrow(i, _):
            zbuf[pl.ds(i * _NL, _NL)] = jnp.zeros((_NL,), jnp.float32)
            return 0
        lax.fori_loop(0, zbuf.shape[0] // _NL, zrow, 0)
        for t in range(c_w // _NL):
            ones_v[pl.ds(t * _NL, _NL)] = jnp.ones((_NL,), jnp.float32)
        pltpu.sync_copy(zbuf.at[pl.ds(0, rpt)], deg_sh.at[pl.ds(rbase, rpt)])
        pltpu.sync_copy(col_hbm.at[w], cidx)
        plsc.subcore_barrier()

        def fire(k_, _):
            pltpu.async_copy(ones_v, deg_sh.at[cidx.at[k_]], sem, add=True)
            return 0
        lax.fori_loop(0, nch, fire, 0)

        def drain(k_, _):
            pltpu.make_async_copy(ones_v, deg_sh.at[cidx.at[0]], sem).wait()
            return 0
        lax.fori_loop(0, nch, drain, 0)
        plsc.subcore_barrier()

        @pl.when(c == 0)
        def _():
            pltpu.sync_copy(deg_sh.at[pl.ds(rbase, rpt)],
                            d0_hbm.at[pl.ds(rbase, rpt)])

        @pl.when(c != 0)
        def _():
            pltpu.sync_copy(deg_sh.at[pl.ds(rbase, rpt)],
                            d1_hbm.at[pl.ds(rbase, rpt)])

    return k(col3d)


def _edge_pass(y, row4d, col4d):
    npad, d = y.shape
    _, nph, cpp, c_w = row4d.shape
    assert cpp > _NB
    ngrp = cpp // _NB
    ntail = cpp - ngrp * _NB
    rpt = npad // _NS
    assert rpt % 8 == 0 and (rpt % c_w) % 8 == 0
    mesh = plsc.VectorSubcoreMesh(core_axis_name="c", subcore_axis_name="s")

    @functools.partial(
        pl.kernel,
        out_type=(jax.ShapeDtypeStruct((npad, d), jnp.float32),
                  jax.ShapeDtypeStruct((npad, d), jnp.float32)),
        mesh=mesh,
        scratch_types=(
            [pltpu.VMEM_SHARED((npad, d), jnp.float32),
             pltpu.VMEM((cpp, c_w), jnp.int32),
             pltpu.VMEM((cpp, c_w), jnp.int32)]
            + [pltpu.VMEM((c_w, d), jnp.float32)] * _NB
            + [pltpu.SemaphoreType.DMA] * (2 * _NB)
        ),
    )
    def k(y_hbm, row_hbm, col_hbm, p0_hbm, p1_hbm,
          acc_sh, ridx, cidx, *bufs_and_sems):
        m = bufs_and_sems[:_NB]
        gs = bufs_and_sems[_NB:2 * _NB]
        ss = bufs_and_sems[2 * _NB:]
        c = lax.axis_index("c")
        s = lax.axis_index("s")
        w = c * _NS + s
        rbase = s * rpt

        @pl.when(c == 0)
        def _():
            pltpu.sync_copy(y_hbm.at[pl.ds(rbase, rpt)],
                            acc_sh.at[pl.ds(rbase, rpt)])

        @pl.when(c != 0)
        def _():
            def zrow(i, _):
                for jj in range(d // _NL):
                    m[0][i, pl.ds(jj * _NL, _NL)] = jnp.zeros((_NL,), jnp.float32)
                return 0
            lax.fori_loop(0, c_w, zrow, 0)
            nfull = rpt // c_w
            for t in range(nfull):
                pltpu.sync_copy(m[0], acc_sh.at[pl.ds(rbase + t * c_w, c_w)])
            rem = rpt - nfull * c_w
            if rem:
                pltpu.sync_copy(
                    m[0].at[pl.ds(0, rem)],
                    acc_sh.at[pl.ds(rbase + nfull * c_w, rem)])

        plsc.subcore_barrier()

        def start(k_, b):
            pltpu.async_copy(y_hbm.at[ridx.at[k_]], m[b], gs[b])

        def wait_g(b):
            pltpu.make_async_copy(y_hbm.at[ridx.at[0]], m[b], gs[b]).wait()

        def scat(k_, b):
            pltpu.async_copy(m[b], acc_sh.at[cidx.at[k_]], ss[b], add=True)

        def wait_s(b):
            pltpu.make_async_copy(m[b], acc_sh.at[cidx.at[0]], ss[b]).wait()

        def phase(p, _):
            pltpu.sync_copy(row_hbm.at[w, p], ridx)
            pltpu.sync_copy(col_hbm.at[w, p], cidx)
            for b in range(_NB):
                start(b, b)

            def group(g, _):
                for b in range(_NB):
                    k_ = g * _NB + b
                    wait_g(b)
                    scat(k_, b)
                for b in range(_NB):
                    kn = (g + 1) * _NB + b

                    @pl.when(kn < cpp)
                    def _():
                        wait_s(b)
                        start(kn, b)
                return 0

            lax.fori_loop(0, ngrp, group, 0)
            for b in range(ntail):
                k_ = ngrp * _NB + b
                wait_g(b)
                scat(k_, b)
            for b in range(_NB):
                wait_s(b)
            return 0

        lax.fori_loop(0, nph, phase, 0)
        plsc.subcore_barrier()

        @pl.when(c == 0)
        def _():
            pltpu.sync_copy(acc_sh.at[pl.ds(rbase, rpt)],
                            p0_hbm.at[pl.ds(rbase, rpt)])

        @pl.when(c != 0)
        def _():
            pltpu.sync_copy(acc_sh.at[pl.ds(rbase, rpt)],
                            p1_hbm.at[pl.ds(rbase, rpt)])

    return k(y, row4d, col4d)


_NROWBLK = 8


def _t1_body(x_ref, w_ref, d0_ref, d1_ref, o_ref):
    dis = lax.rsqrt(d0_ref[...] + d1_ref[...] + 1.0)
    o_ref[...] = dis * jnp.dot(x_ref[...], w_ref[...],
                               preferred_element_type=jnp.float32)


def _t1(x, w, d0, d1):
    npad, d_in = x.shape
    d_out = w.shape[1]
    bm = npad // _NROWBLK
    return pl.pallas_call(
        _t1_body,
        out_shape=jax.ShapeDtypeStruct((npad, d_out), jnp.float32),
        grid=(_NROWBLK,),
        in_specs=[
            pl.BlockSpec((bm, d_in), lambda i: (i, 0)),
            pl.BlockSpec((d_in, d_out), lambda i: (0, 0)),
            pl.BlockSpec((bm, 1), lambda i: (i, 0)),
            pl.BlockSpec((bm, 1), lambda i: (i, 0)),
        ],
        out_specs=pl.BlockSpec((bm, d_out), lambda i: (i, 0)),
    )(x, w, d0, d1)


def _t2_body(p0_ref, p1_ref, d0_ref, d1_ref, b_ref, w_ref, o_ref):
    dis = lax.rsqrt(d0_ref[...] + d1_ref[...] + 1.0)
    h = jnp.maximum(dis * (p0_ref[...] + p1_ref[...]) + b_ref[...], 0.0)
    o_ref[...] = dis * jnp.dot(h, w_ref[...],
                               preferred_element_type=jnp.float32)


def _t2(p0, p1, d0, d1, b, w):
    npad, d = p0.shape
    d_out = w.shape[1]
    bm = npad // _NROWBLK
    return pl.pallas_call(
        _t2_body,
        out_shape=jax.ShapeDtypeStruct((npad, d_out), jnp.float32),
        grid=(_NROWBLK,),
        in_specs=[
            pl.BlockSpec((bm, d), lambda i: (i, 0)),
            pl.BlockSpec((bm, d), lambda i: (i, 0)),
            pl.BlockSpec((bm, 1), lambda i: (i, 0)),
            pl.BlockSpec((bm, 1), lambda i: (i, 0)),
            pl.BlockSpec((1, d), lambda i: (0, 0)),
            pl.BlockSpec((d, d_out), lambda i: (0, 0)),
        ],
        out_specs=pl.BlockSpec((bm, d_out), lambda i: (i, 0)),
    )(p0, p1, d0, d1, b, w)


def _t3_body(q0_ref, q1_ref, d0_ref, d1_ref, b_ref, o_ref):
    dis = lax.rsqrt(d0_ref[...] + d1_ref[...] + 1.0)
    o_ref[...] = dis * (q0_ref[...] + q1_ref[...]) + b_ref[...]


def _t3(q0, q1, d0, d1, b):
    npad, d = q0.shape
    bm = npad // _NROWBLK
    return pl.pallas_call(
        _t3_body,
        out_shape=jax.ShapeDtypeStruct((npad, d), jnp.float32),
        grid=(_NROWBLK,),
        in_specs=[
            pl.BlockSpec((bm, d), lambda i: (i, 0)),
            pl.BlockSpec((bm, d), lambda i: (i, 0)),
            pl.BlockSpec((bm, 1), lambda i: (i, 0)),
            pl.BlockSpec((bm, 1), lambda i: (i, 0)),
            pl.BlockSpec((1, d), lambda i: (0, 0)),
        ],
        out_specs=pl.BlockSpec((bm, d), lambda i: (i, 0)),
    )(q0, q1, d0, d1, b)


def kernel(x, edge_index, W0, b0, W1, b1):
    n, _ = x.shape
    e = edge_index.shape[1]
    assert e % (_NW * _C) == 0, (e, _NW, _C)
    grp = _NS * 8
    npad = ((n + grp - 1) // grp) * grp
    assert npad % (_NROWBLK * 8) == 0

    xpad = jnp.pad(x, ((0, npad - n), (0, 0)))
    nch = e // (_NW * _C)
    nph = 5
    assert nch % nph == 0 and (nch // nph) % 2 == 1
    row4d = edge_index[0].reshape(_NW, nph, nch // nph, _C)
    col4d = edge_index[1].reshape(_NW, nph, nch // nph, _C)

    deg_grp = _NS * 128
    npad_deg = ((n + deg_grp - 1) // deg_grp) * deg_grp
    d0, d1 = _deg_pass(col4d.reshape(_NW, nch, _C), npad_deg)
    d0, d1 = d0[:npad, None], d1[:npad, None]

    y0 = _t1(xpad, W0, d0, d1)
    p0, p1 = _edge_pass(y0, row4d, col4d)
    y1 = _t2(p0, p1, d0, d1, b0[None, :], W1)
    q0, q1 = _edge_pass(y1, row4d, col4d)
    out = _t3(q0, q1, d0, d1, b1[None, :])
    return out[:n]


# speedup vs baseline: 31.8513x; 1.0680x over previous
import functools

import jax
import jax.numpy as jnp
from jax import lax
from jax.experimental import pallas as pl
from jax.experimental.pallas import tpu as pltpu
from jax.experimental.pallas import tpu_sc as plsc

_INFO = plsc.get_sparse_core_info()
_NC = _INFO.num_cores
_NS = _INFO.num_subcores
_NL = _INFO.num_lanes
_NW = _NC * _NS

_C = 80
_NB = 4


def _deg_pass(ei5, npad):
    _, _, nph, cpp, c_w = ei5.shape
    nch = nph * cpp
    rpt = npad // _NS
    mesh = plsc.VectorSubcoreMesh(core_axis_name="c", subcore_axis_name="s")

    @functools.partial(
        pl.kernel,
        out_type=(jax.ShapeDtypeStruct((npad,), jnp.float32),
                  jax.ShapeDtypeStruct((npad,), jnp.float32)),
        mesh=mesh,
        scratch_types=[
            pltpu.VMEM_SHARED((npad,), jnp.float32),
            pltpu.VMEM((nch, c_w), jnp.int32),
            pltpu.VMEM((c_w,), jnp.float32),
            pltpu.VMEM((((rpt + _NL - 1) // _NL) * _NL,), jnp.float32),
            pltpu.SemaphoreType.DMA,
        ],
    )
    def k(ei_hbm, d0_hbm, d1_hbm, deg_sh, cidx, ones_v, zbuf, sem):
        c = lax.axis_index("c")
        s = lax.axis_index("s")
        w = c * _NS + s
        rbase = s * rpt

        def zrow(i, _):
            zbuf[pl.ds(i * _NL, _NL)] = jnp.zeros((_NL,), jnp.float32)
            return 0
        lax.fori_loop(0, zbuf.shape[0] // _NL, zrow, 0)
        for t in range(c_w // _NL):
            ones_v[pl.ds(t * _NL, _NL)] = jnp.ones((_NL,), jnp.float32)
        pltpu.sync_copy(zbuf.at[pl.ds(0, rpt)], deg_sh.at[pl.ds(rbase, rpt)])
        for p in range(nph):
            pltpu.sync_copy(ei_hbm.at[1, w, p],
                            cidx.at[pl.ds(p * cpp, cpp)])
        plsc.subcore_barrier()

        def fire(k_, _):
            pltpu.async_copy(ones_v, deg_sh.at[cidx.at[k_]], sem, add=True)
            return 0
        lax.fori_loop(0, nch, fire, 0)

        def drain(k_, _):
            pltpu.make_async_copy(ones_v, deg_sh.at[cidx.at[0]], sem).wait()
            return 0
        lax.fori_loop(0, nch, drain, 0)
        plsc.subcore_barrier()

        @pl.when(c == 0)
        def _():
            pltpu.sync_copy(deg_sh.at[pl.ds(rbase, rpt)],
                            d0_hbm.at[pl.ds(rbase, rpt)])

        @pl.when(c != 0)
        def _():
            pltpu.sync_copy(deg_sh.at[pl.ds(rbase, rpt)],
                            d1_hbm.at[pl.ds(rbase, rpt)])

    return k(ei5)


def _edge_pass(y, ei5):
    npad, d = y.shape
    _, _, nph, cpp, c_w = ei5.shape
    assert cpp > _NB
    ngrp = cpp // _NB
    ntail = cpp - ngrp * _NB
    rpt = npad // _NS
    assert rpt % 8 == 0 and (rpt % c_w) % 8 == 0
    mesh = plsc.VectorSubcoreMesh(core_axis_name="c", subcore_axis_name="s")

    @functools.partial(
        pl.kernel,
        out_type=(jax.ShapeDtypeStruct((npad, d), jnp.float32),
                  jax.ShapeDtypeStruct((npad, d), jnp.float32)),
        mesh=mesh,
        scratch_types=(
            [pltpu.VMEM_SHARED((npad, d), jnp.float32),
             pltpu.VMEM((cpp, c_w), jnp.int32),
             pltpu.VMEM((cpp, c_w), jnp.int32)]
            + [pltpu.VMEM((c_w, d), jnp.float32)] * _NB
            + [pltpu.SemaphoreType.DMA] * (2 * _NB)
        ),
    )
    def k(y_hbm, ei_hbm, p0_hbm, p1_hbm,
          acc_sh, ridx, cidx, *bufs_and_sems):
        m = bufs_and_sems[:_NB]
        gs = bufs_and_sems[_NB:2 * _NB]
        ss = bufs_and_sems[2 * _NB:]
        c = lax.axis_index("c")
        s = lax.axis_index("s")
        w = c * _NS + s
        rbase = s * rpt

        @pl.when(c == 0)
        def _():
            pltpu.sync_copy(y_hbm.at[pl.ds(rbase, rpt)],
                            acc_sh.at[pl.ds(rbase, rpt)])

        @pl.when(c != 0)
        def _():
            def zrow(i, _):
                for jj in range(d // _NL):
                    m[0][i, pl.ds(jj * _NL, _NL)] = jnp.zeros((_NL,), jnp.float32)
                return 0
            lax.fori_loop(0, c_w, zrow, 0)
            nfull = rpt // c_w
            for t in range(nfull):
                pltpu.sync_copy(m[0], acc_sh.at[pl.ds(rbase + t * c_w, c_w)])
            rem = rpt - nfull * c_w
            if rem:
                pltpu.sync_copy(
                    m[0].at[pl.ds(0, rem)],
                    acc_sh.at[pl.ds(rbase + nfull * c_w, rem)])

        plsc.subcore_barrier()

        def start(k_, b):
            pltpu.async_copy(y_hbm.at[ridx.at[k_]], m[b], gs[b])

        def wait_g(b):
            pltpu.make_async_copy(y_hbm.at[ridx.at[0]], m[b], gs[b]).wait()

        def scat(k_, b):
            pltpu.async_copy(m[b], acc_sh.at[cidx.at[k_]], ss[b], add=True)

        def wait_s(b):
            pltpu.make_async_copy(m[b], acc_sh.at[cidx.at[0]], ss[b]).wait()

        def phase(p, _):
            pltpu.sync_copy(ei_hbm.at[0, w, p], ridx)
            pltpu.sync_copy(ei_hbm.at[1, w, p], cidx)
            for b in range(_NB):
                start(b, b)

            def group(g, _):
                for b in range(_NB):
                    k_ = g * _NB + b
                    wait_g(b)
                    scat(k_, b)
                for b in range(_NB):
                    kn = (g + 1) * _NB + b

                    @pl.when(kn < cpp)
                    def _():
                        wait_s(b)
                        start(kn, b)
                return 0

            lax.fori_loop(0, ngrp, group, 0)
            for b in range(ntail):
                k_ = ngrp * _NB + b
                wait_g(b)
                scat(k_, b)
            for b in range(_NB):
                wait_s(b)
            return 0

        lax.fori_loop(0, nph, phase, 0)
        plsc.subcore_barrier()

        @pl.when(c == 0)
        def _():
            pltpu.sync_copy(acc_sh.at[pl.ds(rbase, rpt)],
                            p0_hbm.at[pl.ds(rbase, rpt)])

        @pl.when(c != 0)
        def _():
            pltpu.sync_copy(acc_sh.at[pl.ds(rbase, rpt)],
                            p1_hbm.at[pl.ds(rbase, rpt)])

    return k(y, ei5)


_NROWBLK = 8


def _t1_body(x_ref, w_ref, dg_ref, o_ref):
    dis = lax.rsqrt(dg_ref[...] + 1.0)
    o_ref[...] = dis * jnp.dot(x_ref[...], w_ref[...],
                               preferred_element_type=jnp.float32)


def _t1(x, w, dg, npad):
    _, d_in = x.shape
    d_out = w.shape[1]
    bm = npad // _NROWBLK
    return pl.pallas_call(
        _t1_body,
        out_shape=jax.ShapeDtypeStruct((npad, d_out), jnp.float32),
        grid=(_NROWBLK,),
        in_specs=[
            pl.BlockSpec((bm, d_in), lambda i: (i, 0)),
            pl.BlockSpec((d_in, d_out), lambda i: (0, 0)),
            pl.BlockSpec((bm, 1), lambda i: (i, 0)),
        ],
        out_specs=pl.BlockSpec((bm, d_out), lambda i: (i, 0)),
    )(x, w, dg)


def _t2_body(p0_ref, p1_ref, dg_ref, b_ref, w_ref, o_ref):
    dis = lax.rsqrt(dg_ref[...] + 1.0)
    h = jnp.maximum(dis * (p0_ref[...] + p1_ref[...]) + b_ref[...], 0.0)
    o_ref[...] = dis * jnp.dot(h, w_ref[...],
                               preferred_element_type=jnp.float32)


def _t2(p0, p1, dg, b, w):
    npad, d = p0.shape
    d_out = w.shape[1]
    bm = npad // _NROWBLK
    return pl.pallas_call(
        _t2_body,
        out_shape=jax.ShapeDtypeStruct((npad, d_out), jnp.float32),
        grid=(_NROWBLK,),
        in_specs=[
            pl.BlockSpec((bm, d), lambda i: (i, 0)),
            pl.BlockSpec((bm, d), lambda i: (i, 0)),
            pl.BlockSpec((bm, 1), lambda i: (i, 0)),
            pl.BlockSpec((1, d), lambda i: (0, 0)),
            pl.BlockSpec((d, d_out), lambda i: (0, 0)),
        ],
        out_specs=pl.BlockSpec((bm, d_out), lambda i: (i, 0)),
    )(p0, p1, dg, b, w)


def _t3_body(q0_ref, q1_ref, dg_ref, b_ref, o_ref):
    dis = lax.rsqrt(dg_ref[...] + 1.0)
    o_ref[...] = dis * (q0_ref[...] + q1_ref[...]) + b_ref[...]


def _t3(q0, q1, dg, b, n):
    npad, d = q0.shape
    bm = npad // _NROWBLK
    return pl.pallas_call(
        _t3_body,
        out_shape=jax.ShapeDtypeStruct((n, d), jnp.float32),
        grid=(_NROWBLK,),
        in_specs=[
            pl.BlockSpec((bm, d), lambda i: (i, 0)),
            pl.BlockSpec((bm, d), lambda i: (i, 0)),
            pl.BlockSpec((bm, 1), lambda i: (i, 0)),
            pl.BlockSpec((1, d), lambda i: (0, 0)),
        ],
        out_specs=pl.BlockSpec((bm, d), lambda i: (i, 0)),
    )(q0, q1, dg, b)


def kernel(x, edge_index, W0, b0, W1, b1):
    n, _ = x.shape
    e = edge_index.shape[1]
    assert e % (_NW * _C) == 0, (e, _NW, _C)
    grp = _NS * 8
    npad = ((n + grp - 1) // grp) * grp
    assert npad % (_NROWBLK * 8) == 0

    nch = e // (_NW * _C)
    nph = 5
    assert nch % nph == 0 and (nch // nph) % 2 == 1
    ei5 = edge_index.reshape(2, _NW, nph, nch // nph, _C)

    deg_grp = _NS * 128
    npad_deg = ((n + deg_grp - 1) // deg_grp) * deg_grp
    d0, d1 = _deg_pass(ei5, npad_deg)
    dg = (d0 + d1)[:npad, None]

    y0 = _t1(x, W0, dg, npad)
    p0, p1 = _edge_pass(y0, ei5)
    y1 = _t2(p0, p1, dg, b0[None, :], W1)
    q0, q1 = _edge_pass(y1, ei5)
    return _t3(q0, q1, dg, b1[None, :], n)
